# Initial kernel scaffold; baseline (speedup 1.0000x reference)
import jax
import jax.numpy as jnp
from jax.experimental import pallas as pl


def kernel(hidden_states, weight, bias):
    raise NotImplementedError("write your pallas kernel here")


# speedup vs baseline: 4.2530x; 4.2530x over previous
import jax
import jax.numpy as jnp
from jax import lax
from jax.experimental import pallas as pl
from jax.experimental.pallas import tpu as pltpu

_EXPERTS = 64
_BT = 512


def _router_body(hs_ref, w_ref, b_ref, scores_ref, idx_ref):
    logits = lax.dot_general(
        hs_ref[...], w_ref[...], (((1,), (1,)), ((), ())),
        preferred_element_type=jnp.float32,
    )
    logits = logits + b_ref[...]
    ex = lax.broadcasted_iota(jnp.int32, logits.shape, 1)
    m1 = jnp.max(logits, axis=1, keepdims=True)
    i1 = jnp.min(jnp.where(logits == m1, ex, _EXPERTS), axis=1, keepdims=True)
    masked = jnp.where(ex == i1, -jnp.inf, logits)
    m2 = jnp.max(masked, axis=1, keepdims=True)
    i2 = jnp.min(jnp.where(masked == m2, ex, _EXPERTS), axis=1, keepdims=True)
    e = jnp.exp(m2 - m1)
    p1 = 1.0 / (1.0 + e)
    p2 = e / (1.0 + e)
    scores_ref[...] = jnp.where(ex == i1, p1, jnp.where(ex == i2, p2, 0.0))
    idx_ref[...] = jnp.concatenate([i1, i2], axis=1)


def kernel(hidden_states, weight, bias):
    tokens, hidden = hidden_states.shape
    grid = (tokens // _BT,)
    scores, idx = pl.pallas_call(
        _router_body,
        grid=grid,
        in_specs=[
            pl.BlockSpec((_BT, hidden), lambda i: (i, 0)),
            pl.BlockSpec((_EXPERTS, hidden), lambda i: (0, 0)),
            pl.BlockSpec((1, _EXPERTS), lambda i: (0, 0)),
        ],
        out_specs=[
            pl.BlockSpec((_BT, _EXPERTS), lambda i: (i, 0)),
            pl.BlockSpec((_BT, 2), lambda i: (i, 0)),
        ],
        out_shape=[
            jax.ShapeDtypeStruct((tokens, _EXPERTS), jnp.float32),
            jax.ShapeDtypeStruct((tokens, 2), jnp.int32),
        ],
    )(hidden_states, weight, bias.reshape(1, _EXPERTS))
    return scores, idx
